# Initial kernel scaffold; baseline (speedup 1.0000x reference)
import jax
import jax.numpy as jnp
from jax.experimental import pallas as pl


def kernel(x, edge_index, edge_attr, W1, b1, W2, b2, W3, b3):
    raise NotImplementedError("write your pallas kernel here")


# speedup vs baseline: 16.6013x; 16.6013x over previous
import functools

import jax
import jax.numpy as jnp
from jax import lax
from jax.experimental import pallas as pl
from jax.experimental.pallas import tpu as pltpu
from jax.experimental.pallas import tpu_sc as plsc

NC, NS, L = 2, 16, 16
NW = NC * NS
GROUP = 125
RT = 640

_f32 = jnp.float32
_i32 = jnp.int32


def _full16(v):
    return jnp.full((L,), v, _i32)


def _zeros16():
    return jnp.zeros((L,), _f32)


def _deg_body(dst2, ew2, degp, dstv, ewv, zbuf, acc, *, rpw):
    c = lax.axis_index("c")
    s = lax.axis_index("s")
    wid = s * NC + c

    def _zb(i, _):
        zbuf[pl.ds(i * L, L)] = _zeros16()
        return 0
    lax.fori_loop(0, RT // L, _zb, 0)
    pltpu.sync_copy(zbuf, acc.at[pl.ds(s * RT, RT)])
    plsc.subcore_barrier()

    pltpu.sync_copy(dst2.at[pl.ds(wid * rpw, rpw)], dstv)
    pltpu.sync_copy(ew2.at[pl.ds(wid * rpw, rpw)], ewv)

    def _row(j, _):
        pltpu.sync_copy(ewv.at[j], acc.at[dstv.at[j]], add=True)
        return 0
    lax.fori_loop(0, rpw, _row, 0)

    plsc.subcore_barrier()
    pltpu.sync_copy(acc.at[pl.ds(s * RT, RT)], degp.at[c, pl.ds(s * RT, RT)])


def _spmv_body(ht, src2, dst2, ewf, out, srcv, dstv, ewv, rows, acc, *, rpw, d):
    c = lax.axis_index("c")
    s = lax.axis_index("s")
    wid = s * NC + c

    def _zr(e, _):
        for k in range(d // L):
            rows[e, pl.ds(k * L, L)] = _zeros16()
        return 0
    lax.fori_loop(0, rows.shape[0], _zr, 0)
    for k in range(RT // rows.shape[0]):
        pltpu.sync_copy(rows, acc.at[pl.ds(s * RT + k * rows.shape[0],
                                           rows.shape[0])])
    plsc.subcore_barrier()

    pltpu.sync_copy(src2.at[pl.ds(wid * rpw, rpw)], srcv)
    pltpu.sync_copy(dst2.at[pl.ds(wid * rpw, rpw)], dstv)
    pltpu.sync_copy(ewf.at[pl.ds(wid * rpw * GROUP, rpw * GROUP)],
                    ewv.at[pl.ds(0, rpw * GROUP)])

    def _batch(j, _):
        pltpu.sync_copy(ht.at[srcv.at[j]], rows.at[pl.ds(0, GROUP)])

        def _mul(e, _):
            w = ewv[pl.ds(j * GROUP + e, L)][0]
            for k in range(d // L):
                sl = pl.ds(k * L, L)
                rows[e, sl] = rows[e, sl] * w
            return 0
        lax.fori_loop(0, GROUP, _mul, 0)

        pltpu.sync_copy(rows.at[pl.ds(0, GROUP)], acc.at[dstv.at[j]], add=True)
        return 0
    lax.fori_loop(0, rpw, _batch, 0)

    plsc.subcore_barrier()
    pltpu.sync_copy(acc.at[pl.ds(s * RT, RT)], out.at[c, pl.ds(s * RT, RT), :])


def _deg_call(dst2, ew2):
    rows2, g = dst2.shape
    rpw = rows2 // NW
    mesh = plsc.VectorSubcoreMesh(core_axis_name="c", subcore_axis_name="s")
    body = functools.partial(_deg_body, rpw=rpw)
    return pl.kernel(
        body,
        out_type=jax.ShapeDtypeStruct((NC, NS * RT), _f32),
        mesh=mesh,
        scratch_types=[
            pltpu.VMEM((rpw, g), _i32),
            pltpu.VMEM((rpw, g), _f32),
            pltpu.VMEM((RT,), _f32),
            pltpu.VMEM_SHARED((NS * RT,), _f32),
        ],
    )(dst2, ew2)


def _spmv_call(ht, src2, dst2, ewf):
    d = ht.shape[1]
    rows2, g = src2.shape
    rpw = rows2 // NW
    mesh = plsc.VectorSubcoreMesh(core_axis_name="c", subcore_axis_name="s")
    body = functools.partial(_spmv_body, rpw=rpw, d=d)
    return pl.kernel(
        body,
        out_type=jax.ShapeDtypeStruct((NC, NS * RT, d), _f32),
        mesh=mesh,
        compiler_params=pltpu.CompilerParams(use_tc_tiling_on_sc=False),
        scratch_types=[
            pltpu.VMEM((rpw, g), _i32),
            pltpu.VMEM((rpw, g), _i32),
            pltpu.VMEM((rpw * g + L,), _f32),
            pltpu.VMEM((128, d), _f32),
            pltpu.VMEM_SHARED((NS * RT, d), _f32),
        ],
    )(ht, src2, dst2, ewf)


def _tc1_body(degp_ref, x_ref, w1_ref, dis_ref, ht1_ref, *, n):
    deg = degp_ref[0] + degp_ref[1] + 1.0
    dis = lax.rsqrt(deg)[:n][:, None]
    dis_ref[...] = dis
    ht1_ref[...] = jnp.dot(x_ref[...], w1_ref[...],
                           preferred_element_type=_f32) * dis


def _tc2_body(s_ref, ht_ref, dis_ref, b_ref, w_ref, h_ref, htn_ref, *, n):
    dis = dis_ref[...]
    S = (s_ref[0] + s_ref[1])[:n]
    h = dis * (S + ht_ref[...]) + b_ref[...][None, :]
    h_ref[...] = h
    htn_ref[...] = jnp.dot(h, w_ref[...], preferred_element_type=_f32) * dis


def _tc3_body(s_ref, ht_ref, dis_ref, b_ref, h1_ref, w3_ref, htn_ref, *, n):
    dis = dis_ref[...]
    S = (s_ref[0] + s_ref[1])[:n]
    h2 = dis * (S + ht_ref[...]) + b_ref[...][None, :]
    dh = h1_ref.shape[1]
    acc = (jnp.dot(h1_ref[...], w3_ref[0:dh], preferred_element_type=_f32)
           + jnp.dot(h2, w3_ref[dh:2 * dh], preferred_element_type=_f32))
    htn_ref[...] = acc * dis


def _tc4_body(s_ref, ht_ref, dis_ref, b_ref, out_ref, *, n):
    S = (s_ref[0] + s_ref[1])[:n]
    h3 = dis_ref[...] * (S + ht_ref[...]) + b_ref[...][None, :]
    m = jnp.max(h3, axis=-1, keepdims=True)
    e = jnp.exp(h3 - m)
    out_ref[...] = e / jnp.sum(e, axis=-1, keepdims=True)


def _tc1(degp, x, w1):
    n = x.shape[0]
    dh = w1.shape[1]
    return pl.pallas_call(
        functools.partial(_tc1_body, n=n),
        out_shape=(jax.ShapeDtypeStruct((n, 1), _f32),
                   jax.ShapeDtypeStruct((n, dh), _f32)),
    )(degp, x, w1)


def _tc2(s, ht, dis, b, w):
    n, dh = ht.shape
    return pl.pallas_call(
        functools.partial(_tc2_body, n=n),
        out_shape=(jax.ShapeDtypeStruct((n, dh), _f32),
                   jax.ShapeDtypeStruct((n, w.shape[1]), _f32)),
    )(s, ht, dis, b, w)


def _tc3(s, ht, dis, b, h1, w3):
    n = ht.shape[0]
    return pl.pallas_call(
        functools.partial(_tc3_body, n=n),
        out_shape=jax.ShapeDtypeStruct((n, w3.shape[1]), _f32),
    )(s, ht, dis, b, h1, w3)


def _tc4(s, ht, dis, b):
    n, do = ht.shape
    return pl.pallas_call(
        functools.partial(_tc4_body, n=n),
        out_shape=jax.ShapeDtypeStruct((n, do), _f32),
    )(s, ht, dis, b)


def kernel(x, edge_index, edge_attr, W1, b1, W2, b2, W3, b3):
    n = x.shape[0]
    e = edge_index.shape[1]
    assert e % (NW * 8 * GROUP) == 0 and n <= NS * RT

    src2 = edge_index[0].reshape(e // GROUP, GROUP).astype(_i32)
    dst2 = edge_index[1].reshape(e // GROUP, GROUP).astype(_i32)
    ew2 = edge_attr.reshape(e // GROUP, GROUP)

    degp = _deg_call(dst2, ew2)
    dis, ht1 = _tc1(degp, x, W1)
    s1 = _spmv_call(ht1, src2, dst2, edge_attr)
    h1, ht2 = _tc2(s1, ht1, dis, b1, W2)
    s2 = _spmv_call(ht2, src2, dst2, edge_attr)
    ht3 = _tc3(s2, ht2, dis, b2, h1, W3)
    s3 = _spmv_call(ht3, src2, dst2, edge_attr)
    return _tc4(s3, ht3, dis, b3)


# speedup vs baseline: 33.1446x; 1.9965x over previous
import functools

import jax
import jax.numpy as jnp
from jax import lax
from jax.experimental import pallas as pl
from jax.experimental.pallas import tpu as pltpu
from jax.experimental.pallas import tpu_sc as plsc

NC, NS, L = 2, 16, 16
NW = NC * NS
GROUP = 125
RT = 640

_f32 = jnp.float32
_i32 = jnp.int32


def _full16(v):
    return jnp.full((L,), v, _i32)


def _zeros16():
    return jnp.zeros((L,), _f32)


def _deg_body(dst2, ew2, degp, dstv, ewv, zbuf, acc, *, rpw):
    c = lax.axis_index("c")
    s = lax.axis_index("s")
    wid = s * NC + c

    def _zb(i, _):
        zbuf[pl.ds(i * L, L)] = _zeros16()
        return 0
    lax.fori_loop(0, RT // L, _zb, 0)
    pltpu.sync_copy(zbuf, acc.at[pl.ds(s * RT, RT)])
    plsc.subcore_barrier()

    pltpu.sync_copy(dst2.at[pl.ds(wid * rpw, rpw)], dstv)
    pltpu.sync_copy(ew2.at[pl.ds(wid * rpw, rpw)], ewv)

    def _row(j, _):
        pltpu.sync_copy(ewv.at[j], acc.at[dstv.at[j]], add=True)
        return 0
    lax.fori_loop(0, rpw, _row, 0)

    plsc.subcore_barrier()
    pltpu.sync_copy(acc.at[pl.ds(s * RT, RT)], degp.at[c, pl.ds(s * RT, RT)])


NBUF = 4


def _spmv_body(ht, src2, dst2, ewf, out, srcv, dstv, ewv,
               rows0, rows1, rows2, rows3, gsems, ssems, acc, *, rpw, d):
    c = lax.axis_index("c")
    s = lax.axis_index("s")
    wid = s * NC + c
    rows = (rows0, rows1, rows2, rows3)

    pltpu.async_copy(src2.at[pl.ds(wid * rpw, rpw)], srcv, gsems.at[0])
    pltpu.async_copy(dst2.at[pl.ds(wid * rpw, rpw)], dstv, gsems.at[1])
    pltpu.async_copy(ewf.at[pl.ds(wid * rpw * GROUP, rpw * GROUP)],
                     ewv.at[pl.ds(0, rpw * GROUP)], gsems.at[2])

    def _zr(e, _):
        for k in range(d // L):
            rows0[e, pl.ds(k * L, L)] = _zeros16()
        return 0
    lax.fori_loop(0, rows0.shape[0], _zr, 0)
    for k in range(RT // rows0.shape[0]):
        pltpu.sync_copy(rows0, acc.at[pl.ds(s * RT + k * rows0.shape[0],
                                            rows0.shape[0])])

    pltpu.make_async_copy(src2.at[pl.ds(wid * rpw, rpw)], srcv,
                          gsems.at[0]).wait()
    pltpu.make_async_copy(dst2.at[pl.ds(wid * rpw, rpw)], dstv,
                          gsems.at[1]).wait()
    pltpu.make_async_copy(ewf.at[pl.ds(wid * rpw * GROUP, rpw * GROUP)],
                          ewv.at[pl.ds(0, rpw * GROUP)], gsems.at[2]).wait()
    plsc.subcore_barrier()

    def _gather(j, b):
        pltpu.async_copy(ht.at[srcv.at[j]], rows[b].at[pl.ds(0, GROUP)],
                         gsems.at[b])

    def _gwait(j, b):
        pltpu.make_async_copy(ht.at[srcv.at[j]], rows[b].at[pl.ds(0, GROUP)],
                              gsems.at[b]).wait()

    def _mul(j, b):
        base = j * GROUP
        buf = rows[b]

        def mbody(e):
            w = ewv[pl.ds(base + e, L)][0]
            for k in range(d // L):
                sl = pl.ds(k * L, L)
                buf[e, sl] = buf[e, sl] * w
        plsc.parallel_loop(0, GROUP, 1, unroll=5)(mbody)

    for b in range(NBUF):
        _gather(b, b)

    def _quad(i, _):
        j = i * NBUF
        descs = []
        for b in range(NBUF):
            _gwait(j + b, b)
            _mul(j + b, b)
            descs.append(pltpu.async_copy(rows[b].at[pl.ds(0, GROUP)],
                                          acc.at[dstv.at[j + b]], ssems.at[b],
                                          add=True))
        for b in range(NBUF):
            descs[b].wait()

            @pl.when(j + b + NBUF < rpw)
            def _():
                _gather(j + b + NBUF, b)
        return 0
    lax.fori_loop(0, rpw // NBUF, _quad, 0)

    plsc.subcore_barrier()
    pltpu.sync_copy(acc.at[pl.ds(s * RT, RT)], out.at[c, pl.ds(s * RT, RT), :])


def _deg_call(dst2, ew2):
    rows2, g = dst2.shape
    rpw = rows2 // NW
    mesh = plsc.VectorSubcoreMesh(core_axis_name="c", subcore_axis_name="s")
    body = functools.partial(_deg_body, rpw=rpw)
    return pl.kernel(
        body,
        out_type=jax.ShapeDtypeStruct((NC, NS * RT), _f32),
        mesh=mesh,
        scratch_types=[
            pltpu.VMEM((rpw, g), _i32),
            pltpu.VMEM((rpw, g), _f32),
            pltpu.VMEM((RT,), _f32),
            pltpu.VMEM_SHARED((NS * RT,), _f32),
        ],
    )(dst2, ew2)


def _spmv_call(ht, src2, dst2, ewf):
    d = ht.shape[1]
    rows2, g = src2.shape
    rpw = rows2 // NW
    mesh = plsc.VectorSubcoreMesh(core_axis_name="c", subcore_axis_name="s")
    body = functools.partial(_spmv_body, rpw=rpw, d=d)
    return pl.kernel(
        body,
        out_type=jax.ShapeDtypeStruct((NC, NS * RT, d), _f32),
        mesh=mesh,
        compiler_params=pltpu.CompilerParams(use_tc_tiling_on_sc=False),
        scratch_types=[
            pltpu.VMEM((rpw, g), _i32),
            pltpu.VMEM((rpw, g), _i32),
            pltpu.VMEM((rpw * g + L,), _f32),
            pltpu.VMEM((128, d), _f32),
            pltpu.VMEM((128, d), _f32),
            pltpu.VMEM((128, d), _f32),
            pltpu.VMEM((128, d), _f32),
            pltpu.SemaphoreType.DMA((NBUF,)),
            pltpu.SemaphoreType.DMA((NBUF,)),
            pltpu.VMEM_SHARED((NS * RT, d), _f32),
        ],
    )(ht, src2, dst2, ewf)


def _tc1_body(degp_ref, x_ref, w1_ref, dis_ref, ht1_ref, *, n):
    deg = degp_ref[0] + degp_ref[1] + 1.0
    dis = lax.rsqrt(deg)[:n][:, None]
    dis_ref[...] = dis
    ht1_ref[...] = jnp.dot(x_ref[...], w1_ref[...],
                           preferred_element_type=_f32) * dis


def _tc2_body(s_ref, ht_ref, dis_ref, b_ref, w_ref, h_ref, htn_ref, *, n):
    dis = dis_ref[...]
    S = (s_ref[0] + s_ref[1])[:n]
    h = dis * (S + ht_ref[...]) + b_ref[...][None, :]
    h_ref[...] = h
    htn_ref[...] = jnp.dot(h, w_ref[...], preferred_element_type=_f32) * dis


def _tc3_body(s_ref, ht_ref, dis_ref, b_ref, h1_ref, w3_ref, htn_ref, *, n):
    dis = dis_ref[...]
    S = (s_ref[0] + s_ref[1])[:n]
    h2 = dis * (S + ht_ref[...]) + b_ref[...][None, :]
    dh = h1_ref.shape[1]
    acc = (jnp.dot(h1_ref[...], w3_ref[0:dh], preferred_element_type=_f32)
           + jnp.dot(h2, w3_ref[dh:2 * dh], preferred_element_type=_f32))
    htn_ref[...] = acc * dis


def _tc4_body(s_ref, ht_ref, dis_ref, b_ref, out_ref, *, n):
    S = (s_ref[0] + s_ref[1])[:n]
    h3 = dis_ref[...] * (S + ht_ref[...]) + b_ref[...][None, :]
    m = jnp.max(h3, axis=-1, keepdims=True)
    e = jnp.exp(h3 - m)
    out_ref[...] = e / jnp.sum(e, axis=-1, keepdims=True)


def _tc1(degp, x, w1):
    n = x.shape[0]
    dh = w1.shape[1]
    return pl.pallas_call(
        functools.partial(_tc1_body, n=n),
        out_shape=(jax.ShapeDtypeStruct((n, 1), _f32),
                   jax.ShapeDtypeStruct((n, dh), _f32)),
    )(degp, x, w1)


def _tc2(s, ht, dis, b, w):
    n, dh = ht.shape
    return pl.pallas_call(
        functools.partial(_tc2_body, n=n),
        out_shape=(jax.ShapeDtypeStruct((n, dh), _f32),
                   jax.ShapeDtypeStruct((n, w.shape[1]), _f32)),
    )(s, ht, dis, b, w)


def _tc3(s, ht, dis, b, h1, w3):
    n = ht.shape[0]
    return pl.pallas_call(
        functools.partial(_tc3_body, n=n),
        out_shape=jax.ShapeDtypeStruct((n, w3.shape[1]), _f32),
    )(s, ht, dis, b, h1, w3)


def _tc4(s, ht, dis, b):
    n, do = ht.shape
    return pl.pallas_call(
        functools.partial(_tc4_body, n=n),
        out_shape=jax.ShapeDtypeStruct((n, do), _f32),
    )(s, ht, dis, b)


def kernel(x, edge_index, edge_attr, W1, b1, W2, b2, W3, b3):
    n = x.shape[0]
    e = edge_index.shape[1]
    assert e % (NW * 8 * GROUP) == 0 and n <= NS * RT

    src2 = edge_index[0].reshape(e // GROUP, GROUP).astype(_i32)
    dst2 = edge_index[1].reshape(e // GROUP, GROUP).astype(_i32)
    ew2 = edge_attr.reshape(e // GROUP, GROUP)

    degp = _deg_call(dst2, ew2)
    dis, ht1 = _tc1(degp, x, W1)
    s1 = _spmv_call(ht1, src2, dst2, edge_attr)
    h1, ht2 = _tc2(s1, ht1, dis, b1, W2)
    s2 = _spmv_call(ht2, src2, dst2, edge_attr)
    ht3 = _tc3(s2, ht2, dis, b2, h1, W3)
    s3 = _spmv_call(ht3, src2, dst2, edge_attr)
    return _tc4(s3, ht3, dis, b3)
